# initial kernel scaffold (unmeasured)
import jax
import jax.numpy as jnp
from jax import lax
from jax.experimental import pallas as pl
from jax.experimental.pallas import tpu as pltpu


def kernel(
    x,
):
    def body(*refs):
        pass

    out_shape = jax.ShapeDtypeStruct(..., jnp.float32)
    return pl.pallas_call(body, out_shape=out_shape)(...)


# baseline (device time: 58969 ns/iter reference)
import jax
import jax.numpy as jnp
from jax import lax
from jax.experimental import pallas as pl
from jax.experimental.pallas import tpu as pltpu

N_DEV = 32
LOG2_N = 5


def kernel(x):
    m_per, n_per = x.shape

    def body(x_ref, out_ref, acc_ref, recv_ref, send_sems, recv_sems):
        my_i = lax.axis_index("i")

        barrier_sem = pltpu.get_barrier_semaphore()
        for r in range(LOG2_N):
            partner = jnp.bitwise_xor(my_i, 2**r)
            pl.semaphore_signal(
                barrier_sem,
                inc=1,
                device_id=(partner,),
                device_id_type=pl.DeviceIdType.MESH,
            )
        pl.semaphore_wait(barrier_sem, LOG2_N)

        xv = x_ref[...]
        m_loc = jnp.max(xv, axis=1, keepdims=True)
        acc_ref[:, 0:1] = m_loc
        e_loc = jnp.exp(xv - m_loc)
        s_loc = jnp.sum(e_loc, axis=1, keepdims=True)
        acc_ref[:, 1:2] = s_loc
        out_ref[...] = e_loc

        for r in range(LOG2_N):
            partner = jnp.bitwise_xor(my_i, 2**r)
            rdma = pltpu.make_async_remote_copy(
                src_ref=acc_ref,
                dst_ref=recv_ref.at[r],
                send_sem=send_sems.at[r],
                recv_sem=recv_sems.at[r],
                device_id=(partner,),
                device_id_type=pl.DeviceIdType.MESH,
            )
            rdma.start()
            rdma.wait()

            m_a = acc_ref[:, 0:1]
            s_a = acc_ref[:, 1:2]
            m_b = recv_ref[r, :, 0:1]
            s_b = recv_ref[r, :, 1:2]
            m_new = jnp.maximum(m_a, m_b)
            s_new = s_a * jnp.exp(m_a - m_new) + s_b * jnp.exp(m_b - m_new)
            acc_ref[:, 0:1] = m_new
            acc_ref[:, 1:2] = s_new

        m_glob = acc_ref[:, 0:1]
        s_glob = acc_ref[:, 1:2]
        out_ref[...] = out_ref[...] * (jnp.exp(m_loc - m_glob) / s_glob)

    return pl.pallas_call(
        body,
        out_shape=jax.ShapeDtypeStruct((m_per, n_per), jnp.float32),
        in_specs=[pl.BlockSpec(memory_space=pltpu.VMEM)],
        out_specs=pl.BlockSpec(memory_space=pltpu.VMEM),
        scratch_shapes=[
            pltpu.VMEM((m_per, 2), jnp.float32),
            pltpu.VMEM((LOG2_N, m_per, 2), jnp.float32),
            pltpu.SemaphoreType.DMA((LOG2_N,)),
            pltpu.SemaphoreType.DMA((LOG2_N,)),
        ],
        compiler_params=pltpu.CompilerParams(collective_id=0),
    )(x)


# device time: 18928 ns/iter; 3.1154x vs baseline; 3.1154x over previous
import jax
import jax.numpy as jnp
from jax import lax
from jax.experimental import pallas as pl
from jax.experimental.pallas import tpu as pltpu

N_DEV = 32
LOG2_N = 5


def kernel(x):
    m_per, n_per = x.shape

    def body(x_ref, out_ref, acc_ref, recv_ref, send_sems, recv_sems):
        my_i = lax.axis_index("i")

        barrier_sem = pltpu.get_barrier_semaphore()
        for r in range(LOG2_N):
            partner = jnp.bitwise_xor(my_i, 2**r)
            pl.semaphore_signal(
                barrier_sem,
                inc=1,
                device_id=(partner,),
                device_id_type=pl.DeviceIdType.MESH,
            )
        pl.semaphore_wait(barrier_sem, LOG2_N)

        xv = x_ref[...]
        m_loc = jnp.max(xv, axis=1, keepdims=True)
        e_loc = jnp.exp(xv - m_loc)
        s_loc = jnp.sum(e_loc, axis=1, keepdims=True)
        acc_ref[...] = jnp.concatenate([m_loc, s_loc], axis=1).T
        out_ref[...] = e_loc

        for r in range(LOG2_N):
            partner = jnp.bitwise_xor(my_i, 2**r)
            rdma = pltpu.make_async_remote_copy(
                src_ref=acc_ref,
                dst_ref=recv_ref.at[r],
                send_sem=send_sems.at[r],
                recv_sem=recv_sems.at[r],
                device_id=(partner,),
                device_id_type=pl.DeviceIdType.MESH,
            )
            rdma.start()
            rdma.wait()

            a = acc_ref[...]
            b = recv_ref[r]
            m_new = jnp.maximum(a[0:1, :], b[0:1, :])
            s_new = a[1:2, :] * jnp.exp(a[0:1, :] - m_new) + b[1:2, :] * jnp.exp(
                b[0:1, :] - m_new
            )
            acc_ref[0:1, :] = m_new
            acc_ref[1:2, :] = s_new

        g = acc_ref[...].T
        out_ref[...] = out_ref[...] * (jnp.exp(m_loc - g[:, 0:1]) / g[:, 1:2])

    return pl.pallas_call(
        body,
        out_shape=jax.ShapeDtypeStruct((m_per, n_per), jnp.float32),
        in_specs=[pl.BlockSpec(memory_space=pltpu.VMEM)],
        out_specs=pl.BlockSpec(memory_space=pltpu.VMEM),
        scratch_shapes=[
            pltpu.VMEM((2, m_per), jnp.float32),
            pltpu.VMEM((LOG2_N, 2, m_per), jnp.float32),
            pltpu.SemaphoreType.DMA((LOG2_N,)),
            pltpu.SemaphoreType.DMA((LOG2_N,)),
        ],
        compiler_params=pltpu.CompilerParams(collective_id=0),
    )(x)
